# baseline (speedup 1.0000x reference)
import functools

import jax
import jax.numpy as jnp
from jax import lax
from jax.experimental import pallas as pl
from jax.experimental.pallas import tpu as pltpu
from jax.experimental.pallas import tpu_sc as plsc

E = 8
TOPK = 2
M = 2048
K = 1024
N = 2048
BM = 256
NB = (M * TOPK) // BM + (E - 1)
PAD_M = NB * BM

NC, NS, LANES = 2, 16, 16
NW = NC * NS
TPW = M // NW
SUB = 16


def _cumsum_log(x, axis):
    n = x.shape[axis]
    sh = 1
    while sh < n:
        zshape = list(x.shape)
        zshape[axis] = sh
        zeros = jnp.zeros(zshape, x.dtype)
        shifted = lax.slice_in_dim(x, 0, n - sh, axis=axis)
        x = x + jnp.concatenate([zeros, shifted], axis=axis)
        sh *= 2
    return x


def _routing_body(idsT_ref, pos_ref, be_ref, act_ref):
    idsT = idsT_ref[...]
    eio = lax.broadcasted_iota(jnp.int32, (E, M), 0)
    oh0 = (idsT[0:1, :] == eio).astype(jnp.int32)
    oh1 = (idsT[1:2, :] == eio).astype(jnp.int32)
    se = oh0 + oh1
    incl = _cumsum_log(se, axis=1)
    excl = incl - se
    counts = incl[:, M - 1:M]
    nblk = (counts + BM - 1) // BM
    blk_incl = _cumsum_log(nblk, axis=0)
    blk_start = blk_incl - nblk
    start = blk_start * BM
    total_blk = blk_incl[E - 1:E, 0:1]

    rank0 = jnp.sum(oh0 * excl, axis=0, keepdims=True)
    rank1 = jnp.sum(oh1 * (excl + oh0), axis=0, keepdims=True)
    base0 = jnp.sum(oh0 * start, axis=0, keepdims=True)
    base1 = jnp.sum(oh1 * start, axis=0, keepdims=True)
    pos_ref[0:1, :] = base0 + rank0
    pos_ref[1:2, :] = base1 + rank1

    bio = lax.broadcasted_iota(jnp.int32, (1, 32), 1)
    bb = jnp.minimum(bio, total_blk - 1)
    be_ref[...] = jnp.sum((blk_start <= bb).astype(jnp.int32), axis=0,
                          keepdims=True) - 1
    act_ref[...] = (bio < total_blk).astype(jnp.int32)


def _routing(idsT):
    return pl.pallas_call(
        _routing_body,
        out_shape=(
            jax.ShapeDtypeStruct((TOPK, M), jnp.int32),
            jax.ShapeDtypeStruct((1, 32), jnp.int32),
            jax.ShapeDtypeStruct((1, 32), jnp.int32),
        ),
    )(idsT)


def _dispatch_body(a1_hbm, pos_hbm, xs_hbm, idx_e, idx_o, x_buf, sem_e, sem_o):
    wid = lax.axis_index("s") * NC + lax.axis_index("c")
    base = wid * TPW
    pltpu.sync_copy(pos_hbm.at[0, pl.ds(base, TPW)], idx_e)
    pltpu.sync_copy(pos_hbm.at[1, pl.ds(base, TPW)], idx_o)
    pltpu.sync_copy(a1_hbm.at[pl.ds(base, TPW)], x_buf)
    cpe = pltpu.make_async_copy(x_buf, xs_hbm.at[idx_e], sem_e)
    cpo = pltpu.make_async_copy(x_buf, xs_hbm.at[idx_o], sem_o)
    cpe.start()
    cpo.start()
    cpe.wait()
    cpo.wait()


def _dispatch(a1, pos):
    mesh = plsc.VectorSubcoreMesh(core_axis_name="c", subcore_axis_name="s")
    f = functools.partial(
        pl.kernel,
        mesh=mesh,
        out_type=jax.ShapeDtypeStruct((PAD_M, K), jnp.float32),
        scratch_types=[
            pltpu.VMEM((TPW,), jnp.int32),
            pltpu.VMEM((TPW,), jnp.int32),
            pltpu.VMEM((TPW, K), jnp.float32),
            pltpu.SemaphoreType.DMA,
            pltpu.SemaphoreType.DMA,
        ],
    )(_dispatch_body)
    return f(a1, pos)


def _gemm_body(be_ref, act_ref, x_ref, w1_ref, w2_ref, out_ref):
    b = pl.program_id(0)

    @pl.when(act_ref[b] == 1)
    def _():
        x = x_ref[...].astype(jnp.bfloat16)
        w1 = w1_ref[0].astype(jnp.bfloat16)
        g = lax.dot_general(x, w1, (((1,), (1,)), ((), ())),
                            preferred_element_type=jnp.float32)
        h = (jax.nn.silu(g[:, : N // 2]) * g[:, N // 2:]).astype(jnp.bfloat16)
        w2 = w2_ref[0].astype(jnp.bfloat16)
        out_ref[...] = lax.dot_general(h, w2, (((1,), (1,)), ((), ())),
                                       preferred_element_type=jnp.float32)


def _grouped_gemm(xs, w1, w2, be, act):
    grid_spec = pltpu.PrefetchScalarGridSpec(
        num_scalar_prefetch=2,
        grid=(NB,),
        in_specs=[
            pl.BlockSpec((BM, K), lambda b, be, act: (b, 0)),
            pl.BlockSpec((1, N, K), lambda b, be, act: (be[b], 0, 0)),
            pl.BlockSpec((1, K, N // 2), lambda b, be, act: (be[b], 0, 0)),
        ],
        out_specs=pl.BlockSpec((BM, K), lambda b, be, act: (b, 0)),
    )
    return pl.pallas_call(
        _gemm_body,
        grid_spec=grid_spec,
        out_shape=jax.ShapeDtypeStruct((PAD_M, K), jnp.float32),
        compiler_params=pltpu.CompilerParams(
            dimension_semantics=("arbitrary",)),
    )(be, act, xs, w1, w2)


def _combine_body(ye_hbm, pos_hbm, twT_hbm, out_hbm,
                  idx_e, idx_o, tw0_v, tw1_v,
                  buf_a0, buf_b0, buf_a1, buf_b1,
                  sem_a0, sem_b0, sem_a1, sem_b1, sem_out):
    wid = lax.axis_index("s") * NC + lax.axis_index("c")
    base = wid * TPW
    pltpu.sync_copy(pos_hbm.at[0, pl.ds(base, TPW)], idx_e)
    pltpu.sync_copy(pos_hbm.at[1, pl.ds(base, TPW)], idx_o)
    pltpu.sync_copy(twT_hbm.at[0, pl.ds(base, TPW)], tw0_v)
    pltpu.sync_copy(twT_hbm.at[1, pl.ds(base, TPW)], tw1_v)
    nsub = TPW // SUB
    bufs = [(buf_a0, buf_b0, sem_a0, sem_b0), (buf_a1, buf_b1, sem_a1, sem_b1)]

    def gather(sub):
        off = sub * SUB
        ba, bb, sa, sb = bufs[sub % 2]
        cpa = pltpu.make_async_copy(ye_hbm.at[idx_e.at[pl.ds(off, SUB)]], ba, sa)
        cpb = pltpu.make_async_copy(ye_hbm.at[idx_o.at[pl.ds(off, SUB)]], bb, sb)
        cpa.start()
        cpb.start()
        return cpa, cpb

    inflight = gather(0)
    out_cp = None
    for sub in range(nsub):
        off = sub * SUB
        ba, bb, _, _ = bufs[sub % 2]
        cpa, cpb = inflight
        cpa.wait()
        cpb.wait()
        if sub + 1 < nsub:
            if out_cp is not None:
                out_cp.wait()
                out_cp = None
            inflight = gather(sub + 1)
        tws0 = tw0_v[pl.ds(off, SUB)]
        tws1 = tw1_v[pl.ds(off, SUB)]

        def row_body(r, carry):
            idx = jnp.broadcast_to(r, (LANES,)).astype(jnp.int32)[:, None]
            dn = lax.GatherDimensionNumbers(
                offset_dims=(), collapsed_slice_dims=(0,), start_index_map=(0,))
            w0 = lax.gather(tws0, idx, dn, (1,),
                            mode=lax.GatherScatterMode.PROMISE_IN_BOUNDS)
            w1v = lax.gather(tws1, idx, dn, (1,),
                             mode=lax.GatherScatterMode.PROMISE_IN_BOUNDS)
            for c in range(K // LANES):
                sl = pl.ds(c * LANES, LANES)
                ba[r, sl] = w0 * ba[r, sl] + w1v * bb[r, sl]
            return carry

        lax.fori_loop(0, SUB, row_body, 0)
        if out_cp is not None:
            out_cp.wait()
        out_cp = pltpu.make_async_copy(
            ba, out_hbm.at[pl.ds(base + off, SUB)], sem_out)
        out_cp.start()
    out_cp.wait()


def _combine(ye, pos, twT):
    mesh = plsc.VectorSubcoreMesh(core_axis_name="c", subcore_axis_name="s")
    f = functools.partial(
        pl.kernel,
        mesh=mesh,
        out_type=jax.ShapeDtypeStruct((M, K), jnp.float32),
        scratch_types=[
            pltpu.VMEM((TPW,), jnp.int32),
            pltpu.VMEM((TPW,), jnp.int32),
            pltpu.VMEM((TPW,), jnp.float32),
            pltpu.VMEM((TPW,), jnp.float32),
            pltpu.VMEM((SUB, K), jnp.float32),
            pltpu.VMEM((SUB, K), jnp.float32),
            pltpu.VMEM((SUB, K), jnp.float32),
            pltpu.VMEM((SUB, K), jnp.float32),
            pltpu.SemaphoreType.DMA,
            pltpu.SemaphoreType.DMA,
            pltpu.SemaphoreType.DMA,
            pltpu.SemaphoreType.DMA,
            pltpu.SemaphoreType.DMA,
        ],
    )(_combine_body)
    return f(ye, pos, twT)


def kernel(a1, w1, w2, topk_weights, topk_ids):
    idsT = topk_ids.astype(jnp.int32).T
    twT = topk_weights.T
    pos, be, act = _routing(idsT)
    xs = _dispatch(a1, pos)
    ye = _grouped_gemm(xs, w1, w2, be.reshape(32), act.reshape(32))
    return _combine(ye, pos, twT)


# speedup vs baseline: 1.0375x; 1.0375x over previous
import functools

import jax
import jax.numpy as jnp
from jax import lax
from jax.experimental import pallas as pl
from jax.experimental.pallas import tpu as pltpu
from jax.experimental.pallas import tpu_sc as plsc

E = 8
TOPK = 2
M = 2048
K = 1024
N = 2048
BM = 512
NB = (M * TOPK) // BM + (E - 1)
PAD_M = NB * BM

NC, NS, LANES = 2, 16, 16
NW = NC * NS
TPW = M // NW
SUB = 16


def _cumsum_log(x, axis):
    n = x.shape[axis]
    sh = 1
    while sh < n:
        zshape = list(x.shape)
        zshape[axis] = sh
        zeros = jnp.zeros(zshape, x.dtype)
        shifted = lax.slice_in_dim(x, 0, n - sh, axis=axis)
        x = x + jnp.concatenate([zeros, shifted], axis=axis)
        sh *= 2
    return x


def _routing_body(idsT_ref, pos_ref, be_ref, act_ref):
    idsT = idsT_ref[...]
    eio = lax.broadcasted_iota(jnp.int32, (E, M), 0)
    oh0 = (idsT[0:1, :] == eio).astype(jnp.int32)
    oh1 = (idsT[1:2, :] == eio).astype(jnp.int32)
    se = oh0 + oh1
    incl = _cumsum_log(se, axis=1)
    excl = incl - se
    counts = incl[:, M - 1:M]
    nblk = (counts + BM - 1) // BM
    blk_incl = _cumsum_log(nblk, axis=0)
    blk_start = blk_incl - nblk
    start = blk_start * BM
    total_blk = blk_incl[E - 1:E, 0:1]

    rank0 = jnp.sum(oh0 * excl, axis=0, keepdims=True)
    rank1 = jnp.sum(oh1 * (excl + oh0), axis=0, keepdims=True)
    base0 = jnp.sum(oh0 * start, axis=0, keepdims=True)
    base1 = jnp.sum(oh1 * start, axis=0, keepdims=True)
    pos_ref[0:1, :] = base0 + rank0
    pos_ref[1:2, :] = base1 + rank1

    bio = lax.broadcasted_iota(jnp.int32, (1, 32), 1)
    bb = jnp.minimum(bio, total_blk - 1)
    be_ref[...] = jnp.sum((blk_start <= bb).astype(jnp.int32), axis=0,
                          keepdims=True) - 1
    act_ref[...] = (bio < total_blk).astype(jnp.int32)


def _routing(idsT):
    return pl.pallas_call(
        _routing_body,
        out_shape=(
            jax.ShapeDtypeStruct((TOPK, M), jnp.int32),
            jax.ShapeDtypeStruct((1, 32), jnp.int32),
            jax.ShapeDtypeStruct((1, 32), jnp.int32),
        ),
    )(idsT)


def _dispatch_body(a1_hbm, pos_hbm, xs_hbm, idx_e, idx_o, x_buf, sem_e, sem_o):
    wid = lax.axis_index("s") * NC + lax.axis_index("c")
    base = wid * TPW
    pltpu.sync_copy(pos_hbm.at[0, pl.ds(base, TPW)], idx_e)
    pltpu.sync_copy(pos_hbm.at[1, pl.ds(base, TPW)], idx_o)
    pltpu.sync_copy(a1_hbm.at[pl.ds(base, TPW)], x_buf)
    cpe = pltpu.make_async_copy(x_buf, xs_hbm.at[idx_e], sem_e)
    cpo = pltpu.make_async_copy(x_buf, xs_hbm.at[idx_o], sem_o)
    cpe.start()
    cpo.start()
    cpe.wait()
    cpo.wait()


def _dispatch(a1, pos):
    mesh = plsc.VectorSubcoreMesh(core_axis_name="c", subcore_axis_name="s")
    f = functools.partial(
        pl.kernel,
        mesh=mesh,
        out_type=jax.ShapeDtypeStruct((PAD_M, K), jnp.float32),
        scratch_types=[
            pltpu.VMEM((TPW,), jnp.int32),
            pltpu.VMEM((TPW,), jnp.int32),
            pltpu.VMEM((TPW, K), jnp.float32),
            pltpu.SemaphoreType.DMA,
            pltpu.SemaphoreType.DMA,
        ],
    )(_dispatch_body)
    return f(a1, pos)


def _gemm_body(be_ref, act_ref, x_ref, w1_ref, w2_ref, out_ref):
    b = pl.program_id(0)

    @pl.when(act_ref[b] == 1)
    def _():
        x = x_ref[...]
        w1 = w1_ref[0]
        g = lax.dot_general(x, w1, (((1,), (1,)), ((), ())),
                            preferred_element_type=jnp.float32)
        h = jax.nn.silu(g[:, : N // 2]) * g[:, N // 2:]
        w2 = w2_ref[0]
        out_ref[...] = lax.dot_general(h, w2, (((1,), (1,)), ((), ())),
                                       preferred_element_type=jnp.float32)


def _grouped_gemm(xs, w1, w2, be, act):
    grid_spec = pltpu.PrefetchScalarGridSpec(
        num_scalar_prefetch=2,
        grid=(NB,),
        in_specs=[
            pl.BlockSpec((BM, K), lambda b, be, act: (b, 0)),
            pl.BlockSpec((1, N, K), lambda b, be, act: (be[b], 0, 0)),
            pl.BlockSpec((1, K, N // 2), lambda b, be, act: (be[b], 0, 0)),
        ],
        out_specs=pl.BlockSpec((BM, K), lambda b, be, act: (b, 0)),
    )
    return pl.pallas_call(
        _gemm_body,
        grid_spec=grid_spec,
        out_shape=jax.ShapeDtypeStruct((PAD_M, K), jnp.float32),
        compiler_params=pltpu.CompilerParams(
            dimension_semantics=("arbitrary",)),
    )(be, act, xs, w1, w2)


def _combine_body(ye_hbm, pos_hbm, twT_hbm, out_hbm,
                  idx_e, idx_o, tw0_v, tw1_v,
                  buf_a0, buf_b0, buf_a1, buf_b1,
                  sem_a0, sem_b0, sem_a1, sem_b1, sem_out):
    wid = lax.axis_index("s") * NC + lax.axis_index("c")
    base = wid * TPW
    pltpu.sync_copy(pos_hbm.at[0, pl.ds(base, TPW)], idx_e)
    pltpu.sync_copy(pos_hbm.at[1, pl.ds(base, TPW)], idx_o)
    pltpu.sync_copy(twT_hbm.at[0, pl.ds(base, TPW)], tw0_v)
    pltpu.sync_copy(twT_hbm.at[1, pl.ds(base, TPW)], tw1_v)
    nsub = TPW // SUB
    bufs = [(buf_a0, buf_b0, sem_a0, sem_b0), (buf_a1, buf_b1, sem_a1, sem_b1)]

    def gather(sub):
        off = sub * SUB
        ba, bb, sa, sb = bufs[sub % 2]
        cpa = pltpu.make_async_copy(ye_hbm.at[idx_e.at[pl.ds(off, SUB)]], ba, sa)
        cpb = pltpu.make_async_copy(ye_hbm.at[idx_o.at[pl.ds(off, SUB)]], bb, sb)
        cpa.start()
        cpb.start()
        return cpa, cpb

    inflight = gather(0)
    out_cp = None
    for sub in range(nsub):
        off = sub * SUB
        ba, bb, _, _ = bufs[sub % 2]
        cpa, cpb = inflight
        cpa.wait()
        cpb.wait()
        if sub + 1 < nsub:
            if out_cp is not None:
                out_cp.wait()
                out_cp = None
            inflight = gather(sub + 1)
        tws0 = tw0_v[pl.ds(off, SUB)]
        tws1 = tw1_v[pl.ds(off, SUB)]

        def row_body(r, carry):
            idx = jnp.broadcast_to(r, (LANES,)).astype(jnp.int32)[:, None]
            dn = lax.GatherDimensionNumbers(
                offset_dims=(), collapsed_slice_dims=(0,), start_index_map=(0,))
            w0 = lax.gather(tws0, idx, dn, (1,),
                            mode=lax.GatherScatterMode.PROMISE_IN_BOUNDS)
            w1v = lax.gather(tws1, idx, dn, (1,),
                             mode=lax.GatherScatterMode.PROMISE_IN_BOUNDS)
            for c in range(K // LANES):
                sl = pl.ds(c * LANES, LANES)
                ba[r, sl] = w0 * ba[r, sl] + w1v * bb[r, sl]
            return carry

        lax.fori_loop(0, SUB, row_body, 0)
        if out_cp is not None:
            out_cp.wait()
        out_cp = pltpu.make_async_copy(
            ba, out_hbm.at[pl.ds(base + off, SUB)], sem_out)
        out_cp.start()
    out_cp.wait()


def _combine(ye, pos, twT):
    mesh = plsc.VectorSubcoreMesh(core_axis_name="c", subcore_axis_name="s")
    f = functools.partial(
        pl.kernel,
        mesh=mesh,
        out_type=jax.ShapeDtypeStruct((M, K), jnp.float32),
        scratch_types=[
            pltpu.VMEM((TPW,), jnp.int32),
            pltpu.VMEM((TPW,), jnp.int32),
            pltpu.VMEM((TPW,), jnp.float32),
            pltpu.VMEM((TPW,), jnp.float32),
            pltpu.VMEM((SUB, K), jnp.float32),
            pltpu.VMEM((SUB, K), jnp.float32),
            pltpu.VMEM((SUB, K), jnp.float32),
            pltpu.VMEM((SUB, K), jnp.float32),
            pltpu.SemaphoreType.DMA,
            pltpu.SemaphoreType.DMA,
            pltpu.SemaphoreType.DMA,
            pltpu.SemaphoreType.DMA,
            pltpu.SemaphoreType.DMA,
        ],
    )(_combine_body)
    return f(ye, pos, twT)


def kernel(a1, w1, w2, topk_weights, topk_ids):
    idsT = topk_ids.astype(jnp.int32).T
    twT = topk_weights.T
    pos, be, act = _routing(idsT)
    xs = _dispatch(a1, pos)
    ye = _grouped_gemm(xs, w1, w2, be.reshape(32), act.reshape(32))
    return _combine(ye, pos, twT)


# speedup vs baseline: 1.1558x; 1.1140x over previous
import functools

import jax
import jax.numpy as jnp
from jax import lax
from jax.experimental import pallas as pl
from jax.experimental.pallas import tpu as pltpu
from jax.experimental.pallas import tpu_sc as plsc

E = 8
TOPK = 2
M = 2048
K = 1024
N = 2048
BM = 640
NB = (M * TOPK) // BM + (E - 1)
PAD_M = NB * BM

NC, NS, LANES = 2, 16, 16
NW = NC * NS
TPW = M // NW
SUB = 16


def _cumsum_log(x, axis):
    n = x.shape[axis]
    sh = 1
    while sh < n:
        zshape = list(x.shape)
        zshape[axis] = sh
        zeros = jnp.zeros(zshape, x.dtype)
        shifted = lax.slice_in_dim(x, 0, n - sh, axis=axis)
        x = x + jnp.concatenate([zeros, shifted], axis=axis)
        sh *= 2
    return x


def _routing_body(idsT_ref, pos_ref, be_ref, act_ref):
    idsT = idsT_ref[...]
    eio = lax.broadcasted_iota(jnp.int32, (E, M), 0)
    oh0 = (idsT[0:1, :] == eio).astype(jnp.int32)
    oh1 = (idsT[1:2, :] == eio).astype(jnp.int32)
    se = oh0 + oh1
    incl = _cumsum_log(se, axis=1)
    excl = incl - se
    counts = incl[:, M - 1:M]
    nblk = (counts + BM - 1) // BM
    blk_incl = _cumsum_log(nblk, axis=0)
    blk_start = blk_incl - nblk
    start = blk_start * BM
    total_blk = blk_incl[E - 1:E, 0:1]

    rank0 = jnp.sum(oh0 * excl, axis=0, keepdims=True)
    rank1 = jnp.sum(oh1 * (excl + oh0), axis=0, keepdims=True)
    base0 = jnp.sum(oh0 * start, axis=0, keepdims=True)
    base1 = jnp.sum(oh1 * start, axis=0, keepdims=True)
    pos_ref[0:1, :] = base0 + rank0
    pos_ref[1:2, :] = base1 + rank1

    bio = lax.broadcasted_iota(jnp.int32, (1, 32), 1)
    bb = jnp.minimum(bio, total_blk - 1)
    be_ref[...] = jnp.sum((blk_start <= bb).astype(jnp.int32), axis=0,
                          keepdims=True) - 1
    act_ref[...] = (bio < total_blk).astype(jnp.int32)


def _routing(idsT):
    return pl.pallas_call(
        _routing_body,
        out_shape=(
            jax.ShapeDtypeStruct((TOPK, M), jnp.int32),
            jax.ShapeDtypeStruct((1, 32), jnp.int32),
            jax.ShapeDtypeStruct((1, 32), jnp.int32),
        ),
    )(idsT)


def _dispatch_body(a1_hbm, pos_hbm, xs_hbm, idx_e, idx_o, x_buf, sem_e, sem_o):
    wid = lax.axis_index("s") * NC + lax.axis_index("c")
    base = wid * TPW
    pltpu.sync_copy(pos_hbm.at[0, pl.ds(base, TPW)], idx_e)
    pltpu.sync_copy(pos_hbm.at[1, pl.ds(base, TPW)], idx_o)
    pltpu.sync_copy(a1_hbm.at[pl.ds(base, TPW)], x_buf)
    cpe = pltpu.make_async_copy(x_buf, xs_hbm.at[idx_e], sem_e)
    cpo = pltpu.make_async_copy(x_buf, xs_hbm.at[idx_o], sem_o)
    cpe.start()
    cpo.start()
    cpe.wait()
    cpo.wait()


def _dispatch(a1, pos):
    mesh = plsc.VectorSubcoreMesh(core_axis_name="c", subcore_axis_name="s")
    f = functools.partial(
        pl.kernel,
        mesh=mesh,
        out_type=jax.ShapeDtypeStruct((PAD_M, K), jnp.float32),
        scratch_types=[
            pltpu.VMEM((TPW,), jnp.int32),
            pltpu.VMEM((TPW,), jnp.int32),
            pltpu.VMEM((TPW, K), jnp.float32),
            pltpu.SemaphoreType.DMA,
            pltpu.SemaphoreType.DMA,
        ],
    )(_dispatch_body)
    return f(a1, pos)


def _gemm_body(be_ref, act_ref, x_ref, w1_ref, w2_ref, out_ref):
    b = pl.program_id(0)

    @pl.when(act_ref[b] == 1)
    def _():
        x = x_ref[...]
        w1 = w1_ref[0]
        g = lax.dot_general(x, w1, (((1,), (1,)), ((), ())),
                            preferred_element_type=jnp.float32)
        h = jax.nn.silu(g[:, : N // 2]) * g[:, N // 2:]
        w2 = w2_ref[0]
        out_ref[...] = lax.dot_general(h, w2, (((1,), (1,)), ((), ())),
                                       preferred_element_type=jnp.float32)


def _grouped_gemm(xs, w1, w2, be, act):
    grid_spec = pltpu.PrefetchScalarGridSpec(
        num_scalar_prefetch=2,
        grid=(NB,),
        in_specs=[
            pl.BlockSpec((BM, K), lambda b, be, act: (b, 0)),
            pl.BlockSpec((1, N, K), lambda b, be, act: (be[b], 0, 0)),
            pl.BlockSpec((1, K, N // 2), lambda b, be, act: (be[b], 0, 0)),
        ],
        out_specs=pl.BlockSpec((BM, K), lambda b, be, act: (b, 0)),
    )
    return pl.pallas_call(
        _gemm_body,
        grid_spec=grid_spec,
        out_shape=jax.ShapeDtypeStruct((PAD_M, K), jnp.float32),
        compiler_params=pltpu.CompilerParams(
            dimension_semantics=("arbitrary",)),
    )(be, act, xs, w1, w2)


def _combine_body(ye_hbm, pos_hbm, twT_hbm, out_hbm,
                  idx_e, idx_o, tw0_v, tw1_v,
                  buf_a0, buf_b0, buf_a1, buf_b1,
                  sem_a0, sem_b0, sem_a1, sem_b1, sem_out):
    wid = lax.axis_index("s") * NC + lax.axis_index("c")
    base = wid * TPW
    pltpu.sync_copy(pos_hbm.at[0, pl.ds(base, TPW)], idx_e)
    pltpu.sync_copy(pos_hbm.at[1, pl.ds(base, TPW)], idx_o)
    pltpu.sync_copy(twT_hbm.at[0, pl.ds(base, TPW)], tw0_v)
    pltpu.sync_copy(twT_hbm.at[1, pl.ds(base, TPW)], tw1_v)
    nsub = TPW // SUB
    bufs = [(buf_a0, buf_b0, sem_a0, sem_b0), (buf_a1, buf_b1, sem_a1, sem_b1)]

    def gather(sub):
        off = sub * SUB
        ba, bb, sa, sb = bufs[sub % 2]
        cpa = pltpu.make_async_copy(ye_hbm.at[idx_e.at[pl.ds(off, SUB)]], ba, sa)
        cpb = pltpu.make_async_copy(ye_hbm.at[idx_o.at[pl.ds(off, SUB)]], bb, sb)
        cpa.start()
        cpb.start()
        return cpa, cpb

    inflight = gather(0)
    out_cp = None
    for sub in range(nsub):
        off = sub * SUB
        ba, bb, _, _ = bufs[sub % 2]
        cpa, cpb = inflight
        cpa.wait()
        cpb.wait()
        if sub + 1 < nsub:
            if out_cp is not None:
                out_cp.wait()
                out_cp = None
            inflight = gather(sub + 1)
        tws0 = tw0_v[pl.ds(off, SUB)]
        tws1 = tw1_v[pl.ds(off, SUB)]

        def row_body(r, carry):
            idx = jnp.broadcast_to(r, (LANES,)).astype(jnp.int32)[:, None]
            dn = lax.GatherDimensionNumbers(
                offset_dims=(), collapsed_slice_dims=(0,), start_index_map=(0,))
            w0 = lax.gather(tws0, idx, dn, (1,),
                            mode=lax.GatherScatterMode.PROMISE_IN_BOUNDS)
            w1v = lax.gather(tws1, idx, dn, (1,),
                             mode=lax.GatherScatterMode.PROMISE_IN_BOUNDS)
            for c in range(K // LANES):
                sl = pl.ds(c * LANES, LANES)
                ba[r, sl] = w0 * ba[r, sl] + w1v * bb[r, sl]
            return carry

        lax.fori_loop(0, SUB, row_body, 0)
        if out_cp is not None:
            out_cp.wait()
        out_cp = pltpu.make_async_copy(
            ba, out_hbm.at[pl.ds(base + off, SUB)], sem_out)
        out_cp.start()
    out_cp.wait()


def _combine(ye, pos, twT):
    mesh = plsc.VectorSubcoreMesh(core_axis_name="c", subcore_axis_name="s")
    f = functools.partial(
        pl.kernel,
        mesh=mesh,
        out_type=jax.ShapeDtypeStruct((M, K), jnp.float32),
        scratch_types=[
            pltpu.VMEM((TPW,), jnp.int32),
            pltpu.VMEM((TPW,), jnp.int32),
            pltpu.VMEM((TPW,), jnp.float32),
            pltpu.VMEM((TPW,), jnp.float32),
            pltpu.VMEM((SUB, K), jnp.float32),
            pltpu.VMEM((SUB, K), jnp.float32),
            pltpu.VMEM((SUB, K), jnp.float32),
            pltpu.VMEM((SUB, K), jnp.float32),
            pltpu.SemaphoreType.DMA,
            pltpu.SemaphoreType.DMA,
            pltpu.SemaphoreType.DMA,
            pltpu.SemaphoreType.DMA,
            pltpu.SemaphoreType.DMA,
        ],
    )(_combine_body)
    return f(ye, pos, twT)


def kernel(a1, w1, w2, topk_weights, topk_ids):
    idsT = topk_ids.astype(jnp.int32).T
    twT = topk_weights.T
    pos, be, act = _routing(idsT)
    xs = _dispatch(a1, pos)
    ye = _grouped_gemm(xs, w1, w2, be.reshape(32), act.reshape(32))
    return _combine(ye, pos, twT)


# speedup vs baseline: 1.1632x; 1.0064x over previous
import functools

import jax
import jax.numpy as jnp
from jax import lax
from jax.experimental import pallas as pl
from jax.experimental.pallas import tpu as pltpu
from jax.experimental.pallas import tpu_sc as plsc

E = 8
TOPK = 2
M = 2048
K = 1024
N = 2048
BM = 576
NB = (M * TOPK) // BM + (E - 1)
PAD_M = NB * BM

NC, NS, LANES = 2, 16, 16
NW = NC * NS
TPW = M // NW
SUB = 16


def _cumsum_log(x, axis):
    n = x.shape[axis]
    sh = 1
    while sh < n:
        zshape = list(x.shape)
        zshape[axis] = sh
        zeros = jnp.zeros(zshape, x.dtype)
        shifted = lax.slice_in_dim(x, 0, n - sh, axis=axis)
        x = x + jnp.concatenate([zeros, shifted], axis=axis)
        sh *= 2
    return x


def _routing_body(idsT_ref, pos_ref, be_ref, act_ref):
    idsT = idsT_ref[...]
    eio = lax.broadcasted_iota(jnp.int32, (E, M), 0)
    oh0 = (idsT[0:1, :] == eio).astype(jnp.int32)
    oh1 = (idsT[1:2, :] == eio).astype(jnp.int32)
    se = oh0 + oh1
    incl = _cumsum_log(se, axis=1)
    excl = incl - se
    counts = incl[:, M - 1:M]
    nblk = (counts + BM - 1) // BM
    blk_incl = _cumsum_log(nblk, axis=0)
    blk_start = blk_incl - nblk
    start = blk_start * BM
    total_blk = blk_incl[E - 1:E, 0:1]

    rank0 = jnp.sum(oh0 * excl, axis=0, keepdims=True)
    rank1 = jnp.sum(oh1 * (excl + oh0), axis=0, keepdims=True)
    base0 = jnp.sum(oh0 * start, axis=0, keepdims=True)
    base1 = jnp.sum(oh1 * start, axis=0, keepdims=True)
    pos_ref[0:1, :] = base0 + rank0
    pos_ref[1:2, :] = base1 + rank1

    bio = lax.broadcasted_iota(jnp.int32, (1, 32), 1)
    bb = jnp.minimum(bio, total_blk - 1)
    be_ref[...] = jnp.sum((blk_start <= bb).astype(jnp.int32), axis=0,
                          keepdims=True) - 1
    act_ref[...] = (bio < total_blk).astype(jnp.int32)


def _routing(idsT):
    return pl.pallas_call(
        _routing_body,
        out_shape=(
            jax.ShapeDtypeStruct((TOPK, M), jnp.int32),
            jax.ShapeDtypeStruct((1, 32), jnp.int32),
            jax.ShapeDtypeStruct((1, 32), jnp.int32),
        ),
    )(idsT)


def _dispatch_body(a1_hbm, pos_hbm, xs_hbm, idx_e, idx_o, x_buf, sem_e, sem_o):
    wid = lax.axis_index("s") * NC + lax.axis_index("c")
    base = wid * TPW
    pltpu.sync_copy(pos_hbm.at[0, pl.ds(base, TPW)], idx_e)
    pltpu.sync_copy(pos_hbm.at[1, pl.ds(base, TPW)], idx_o)
    pltpu.sync_copy(a1_hbm.at[pl.ds(base, TPW)], x_buf)
    cpe = pltpu.make_async_copy(x_buf, xs_hbm.at[idx_e], sem_e)
    cpo = pltpu.make_async_copy(x_buf, xs_hbm.at[idx_o], sem_o)
    cpe.start()
    cpo.start()
    cpe.wait()
    cpo.wait()


def _dispatch(a1, pos):
    mesh = plsc.VectorSubcoreMesh(core_axis_name="c", subcore_axis_name="s")
    f = functools.partial(
        pl.kernel,
        mesh=mesh,
        out_type=jax.ShapeDtypeStruct((PAD_M, K), jnp.float32),
        scratch_types=[
            pltpu.VMEM((TPW,), jnp.int32),
            pltpu.VMEM((TPW,), jnp.int32),
            pltpu.VMEM((TPW, K), jnp.float32),
            pltpu.SemaphoreType.DMA,
            pltpu.SemaphoreType.DMA,
        ],
    )(_dispatch_body)
    return f(a1, pos)


def _gemm_body(be_ref, act_ref, x_ref, w1_ref, w2_ref, out_ref):
    b = pl.program_id(0)

    @pl.when(act_ref[b] == 1)
    def _():
        x = x_ref[...]
        w1 = w1_ref[0]
        g = lax.dot_general(x, w1, (((1,), (1,)), ((), ())),
                            preferred_element_type=jnp.float32)
        h = jax.nn.silu(g[:, : N // 2]) * g[:, N // 2:]
        w2 = w2_ref[0]
        out_ref[...] = lax.dot_general(h, w2, (((1,), (1,)), ((), ())),
                                       preferred_element_type=jnp.float32)


def _grouped_gemm(xs, w1, w2, be, act):
    grid_spec = pltpu.PrefetchScalarGridSpec(
        num_scalar_prefetch=2,
        grid=(NB,),
        in_specs=[
            pl.BlockSpec((BM, K), lambda b, be, act: (b, 0)),
            pl.BlockSpec((1, N, K), lambda b, be, act: (be[b], 0, 0)),
            pl.BlockSpec((1, K, N // 2), lambda b, be, act: (be[b], 0, 0)),
        ],
        out_specs=pl.BlockSpec((BM, K), lambda b, be, act: (b, 0)),
    )
    return pl.pallas_call(
        _gemm_body,
        grid_spec=grid_spec,
        out_shape=jax.ShapeDtypeStruct((PAD_M, K), jnp.float32),
        compiler_params=pltpu.CompilerParams(
            dimension_semantics=("arbitrary",)),
    )(be, act, xs, w1, w2)


def _combine_body(ye_hbm, pos_hbm, twT_hbm, out_hbm,
                  idx_e, idx_o, tw0_v, tw1_v,
                  buf_a0, buf_b0, buf_a1, buf_b1,
                  sem_a0, sem_b0, sem_a1, sem_b1, sem_out):
    wid = lax.axis_index("s") * NC + lax.axis_index("c")
    base = wid * TPW
    pltpu.sync_copy(pos_hbm.at[0, pl.ds(base, TPW)], idx_e)
    pltpu.sync_copy(pos_hbm.at[1, pl.ds(base, TPW)], idx_o)
    pltpu.sync_copy(twT_hbm.at[0, pl.ds(base, TPW)], tw0_v)
    pltpu.sync_copy(twT_hbm.at[1, pl.ds(base, TPW)], tw1_v)
    nsub = TPW // SUB
    bufs = [(buf_a0, buf_b0, sem_a0, sem_b0), (buf_a1, buf_b1, sem_a1, sem_b1)]

    def gather(sub):
        off = sub * SUB
        ba, bb, sa, sb = bufs[sub % 2]
        cpa = pltpu.make_async_copy(ye_hbm.at[idx_e.at[pl.ds(off, SUB)]], ba, sa)
        cpb = pltpu.make_async_copy(ye_hbm.at[idx_o.at[pl.ds(off, SUB)]], bb, sb)
        cpa.start()
        cpb.start()
        return cpa, cpb

    inflight = gather(0)
    out_cp = None
    for sub in range(nsub):
        off = sub * SUB
        ba, bb, _, _ = bufs[sub % 2]
        cpa, cpb = inflight
        cpa.wait()
        cpb.wait()
        if sub + 1 < nsub:
            if out_cp is not None:
                out_cp.wait()
                out_cp = None
            inflight = gather(sub + 1)
        tws0 = tw0_v[pl.ds(off, SUB)]
        tws1 = tw1_v[pl.ds(off, SUB)]

        def row_body(r, carry):
            idx = jnp.broadcast_to(r, (LANES,)).astype(jnp.int32)[:, None]
            dn = lax.GatherDimensionNumbers(
                offset_dims=(), collapsed_slice_dims=(0,), start_index_map=(0,))
            w0 = lax.gather(tws0, idx, dn, (1,),
                            mode=lax.GatherScatterMode.PROMISE_IN_BOUNDS)
            w1v = lax.gather(tws1, idx, dn, (1,),
                             mode=lax.GatherScatterMode.PROMISE_IN_BOUNDS)
            for c in range(K // LANES):
                sl = pl.ds(c * LANES, LANES)
                ba[r, sl] = w0 * ba[r, sl] + w1v * bb[r, sl]
            return carry

        lax.fori_loop(0, SUB, row_body, 0)
        if out_cp is not None:
            out_cp.wait()
        out_cp = pltpu.make_async_copy(
            ba, out_hbm.at[pl.ds(base + off, SUB)], sem_out)
        out_cp.start()
    out_cp.wait()


def _combine(ye, pos, twT):
    mesh = plsc.VectorSubcoreMesh(core_axis_name="c", subcore_axis_name="s")
    f = functools.partial(
        pl.kernel,
        mesh=mesh,
        out_type=jax.ShapeDtypeStruct((M, K), jnp.float32),
        scratch_types=[
            pltpu.VMEM((TPW,), jnp.int32),
            pltpu.VMEM((TPW,), jnp.int32),
            pltpu.VMEM((TPW,), jnp.float32),
            pltpu.VMEM((TPW,), jnp.float32),
            pltpu.VMEM((SUB, K), jnp.float32),
            pltpu.VMEM((SUB, K), jnp.float32),
            pltpu.VMEM((SUB, K), jnp.float32),
            pltpu.VMEM((SUB, K), jnp.float32),
            pltpu.SemaphoreType.DMA,
            pltpu.SemaphoreType.DMA,
            pltpu.SemaphoreType.DMA,
            pltpu.SemaphoreType.DMA,
            pltpu.SemaphoreType.DMA,
        ],
    )(_combine_body)
    return f(ye, pos, twT)


def kernel(a1, w1, w2, topk_weights, topk_ids):
    idsT = topk_ids.astype(jnp.int32).T
    twT = topk_weights.T
    pos, be, act = _routing(idsT)
    xs = _dispatch(a1, pos)
    ye = _grouped_gemm(xs, w1, w2, be.reshape(32), act.reshape(32))
    return _combine(ye, pos, twT)


# speedup vs baseline: 1.2737x; 1.0951x over previous
import functools

import jax
import jax.numpy as jnp
from jax import lax
from jax.experimental import pallas as pl
from jax.experimental.pallas import tpu as pltpu
from jax.experimental.pallas import tpu_sc as plsc

E = 8
TOPK = 2
M = 2048
K = 1024
N = 2048
BM = 576
NB = (M * TOPK) // BM + (E - 1)
PAD_M = NB * BM

NC, NS, LANES = 2, 16, 16
NW = NC * NS
TPW = M // NW
SUB = 16


def _cumsum_log(x, axis):
    n = x.shape[axis]
    sh = 1
    while sh < n:
        zshape = list(x.shape)
        zshape[axis] = sh
        zeros = jnp.zeros(zshape, x.dtype)
        shifted = lax.slice_in_dim(x, 0, n - sh, axis=axis)
        x = x + jnp.concatenate([zeros, shifted], axis=axis)
        sh *= 2
    return x


def _routing_body(idsT_ref, pos_ref, be_ref, act_ref, bx_ref):
    idsT = idsT_ref[...]
    eio = lax.broadcasted_iota(jnp.int32, (E, M), 0)
    oh0 = (idsT[0:1, :] == eio).astype(jnp.int32)
    oh1 = (idsT[1:2, :] == eio).astype(jnp.int32)
    se = oh0 + oh1
    incl = _cumsum_log(se, axis=1)
    excl = incl - se
    counts = incl[:, M - 1:M]
    nblk = (counts + BM - 1) // BM
    blk_incl = _cumsum_log(nblk, axis=0)
    blk_start = blk_incl - nblk
    start = blk_start * BM
    total_blk = blk_incl[E - 1:E, 0:1]

    rank0 = jnp.sum(oh0 * excl, axis=0, keepdims=True)
    rank1 = jnp.sum(oh1 * (excl + oh0), axis=0, keepdims=True)
    base0 = jnp.sum(oh0 * start, axis=0, keepdims=True)
    base1 = jnp.sum(oh1 * start, axis=0, keepdims=True)
    pos_ref[0:1, :] = base0 + rank0
    pos_ref[1:2, :] = base1 + rank1

    bio = lax.broadcasted_iota(jnp.int32, (1, 32), 1)
    bb = jnp.minimum(bio, total_blk - 1)
    be_ref[...] = jnp.sum((blk_start <= bb).astype(jnp.int32), axis=0,
                          keepdims=True) - 1
    act_ref[...] = (bio < total_blk).astype(jnp.int32)
    bx_ref[...] = bb


def _routing(idsT):
    return pl.pallas_call(
        _routing_body,
        out_shape=(
            jax.ShapeDtypeStruct((TOPK, M), jnp.int32),
            jax.ShapeDtypeStruct((1, 32), jnp.int32),
            jax.ShapeDtypeStruct((1, 32), jnp.int32),
            jax.ShapeDtypeStruct((1, 32), jnp.int32),
        ),
    )(idsT)


def _dispatch_body(a1_hbm, pos_hbm, xs_hbm, idx_e, idx_o, x_buf, sem_e, sem_o):
    wid = lax.axis_index("s") * NC + lax.axis_index("c")
    base = wid * TPW
    pltpu.sync_copy(pos_hbm.at[0, pl.ds(base, TPW)], idx_e)
    pltpu.sync_copy(pos_hbm.at[1, pl.ds(base, TPW)], idx_o)
    pltpu.sync_copy(a1_hbm.at[pl.ds(base, TPW)], x_buf)
    cpe = pltpu.make_async_copy(x_buf, xs_hbm.at[idx_e], sem_e)
    cpo = pltpu.make_async_copy(x_buf, xs_hbm.at[idx_o], sem_o)
    cpe.start()
    cpo.start()
    cpe.wait()
    cpo.wait()


def _dispatch(a1, pos):
    mesh = plsc.VectorSubcoreMesh(core_axis_name="c", subcore_axis_name="s")
    f = functools.partial(
        pl.kernel,
        mesh=mesh,
        out_type=jax.ShapeDtypeStruct((PAD_M, K), jnp.float32),
        scratch_types=[
            pltpu.VMEM((TPW,), jnp.int32),
            pltpu.VMEM((TPW,), jnp.int32),
            pltpu.VMEM((TPW, K), jnp.float32),
            pltpu.SemaphoreType.DMA,
            pltpu.SemaphoreType.DMA,
        ],
    )(_dispatch_body)
    return f(a1, pos)


def _gemm_body(be_ref, act_ref, bx_ref, x_ref, w1_ref, w2_ref, out_ref):
    b = pl.program_id(0)

    @pl.when(act_ref[b] == 1)
    def _():
        x = x_ref[...]
        w1 = w1_ref[0]
        g = lax.dot_general(x, w1, (((1,), (1,)), ((), ())),
                            preferred_element_type=jnp.float32)
        h = jax.nn.silu(g[:, : N // 2]) * g[:, N // 2:]
        w2 = w2_ref[0]
        out_ref[...] = lax.dot_general(h, w2, (((1,), (1,)), ((), ())),
                                       preferred_element_type=jnp.float32)


def _grouped_gemm(xs, w1, w2, be, act, bx):
    grid_spec = pltpu.PrefetchScalarGridSpec(
        num_scalar_prefetch=3,
        grid=(NB,),
        in_specs=[
            pl.BlockSpec((BM, K), lambda b, be, act, bx: (bx[b], 0)),
            pl.BlockSpec((1, N, K), lambda b, be, act, bx: (be[b], 0, 0)),
            pl.BlockSpec((1, K, N // 2),
                         lambda b, be, act, bx: (be[b], 0, 0)),
        ],
        out_specs=pl.BlockSpec((BM, K), lambda b, be, act, bx: (bx[b], 0)),
    )
    return pl.pallas_call(
        _gemm_body,
        grid_spec=grid_spec,
        out_shape=jax.ShapeDtypeStruct((PAD_M, K), jnp.float32),
        compiler_params=pltpu.CompilerParams(
            dimension_semantics=("arbitrary",)),
    )(be, act, bx, xs, w1, w2)


def _combine_body(ye_hbm, pos_hbm, twT_hbm, out_hbm,
                  idx_e, idx_o, tw0_v, tw1_v,
                  buf_a0, buf_b0, buf_a1, buf_b1,
                  sem_a0, sem_b0, sem_a1, sem_b1, sem_out):
    wid = lax.axis_index("s") * NC + lax.axis_index("c")
    base = wid * TPW
    pltpu.sync_copy(pos_hbm.at[0, pl.ds(base, TPW)], idx_e)
    pltpu.sync_copy(pos_hbm.at[1, pl.ds(base, TPW)], idx_o)
    pltpu.sync_copy(twT_hbm.at[0, pl.ds(base, TPW)], tw0_v)
    pltpu.sync_copy(twT_hbm.at[1, pl.ds(base, TPW)], tw1_v)
    nsub = TPW // SUB
    bufs = [(buf_a0, buf_b0, sem_a0, sem_b0), (buf_a1, buf_b1, sem_a1, sem_b1)]

    def gather(sub):
        off = sub * SUB
        ba, bb, sa, sb = bufs[sub % 2]
        cpa = pltpu.make_async_copy(ye_hbm.at[idx_e.at[pl.ds(off, SUB)]], ba, sa)
        cpb = pltpu.make_async_copy(ye_hbm.at[idx_o.at[pl.ds(off, SUB)]], bb, sb)
        cpa.start()
        cpb.start()
        return cpa, cpb

    inflight = gather(0)
    out_cp = None
    for sub in range(nsub):
        off = sub * SUB
        ba, bb, _, _ = bufs[sub % 2]
        cpa, cpb = inflight
        cpa.wait()
        cpb.wait()
        if sub + 1 < nsub:
            if out_cp is not None:
                out_cp.wait()
                out_cp = None
            inflight = gather(sub + 1)
        tws0 = tw0_v[pl.ds(off, SUB)]
        tws1 = tw1_v[pl.ds(off, SUB)]

        def row_body(r, carry):
            idx = jnp.broadcast_to(r, (LANES,)).astype(jnp.int32)[:, None]
            dn = lax.GatherDimensionNumbers(
                offset_dims=(), collapsed_slice_dims=(0,), start_index_map=(0,))
            w0 = lax.gather(tws0, idx, dn, (1,),
                            mode=lax.GatherScatterMode.PROMISE_IN_BOUNDS)
            w1v = lax.gather(tws1, idx, dn, (1,),
                             mode=lax.GatherScatterMode.PROMISE_IN_BOUNDS)
            for c in range(K // LANES):
                sl = pl.ds(c * LANES, LANES)
                ba[r, sl] = w0 * ba[r, sl] + w1v * bb[r, sl]
            return carry

        lax.fori_loop(0, SUB, row_body, 0)
        if out_cp is not None:
            out_cp.wait()
        out_cp = pltpu.make_async_copy(
            ba, out_hbm.at[pl.ds(base + off, SUB)], sem_out)
        out_cp.start()
    out_cp.wait()


def _combine(ye, pos, twT):
    mesh = plsc.VectorSubcoreMesh(core_axis_name="c", subcore_axis_name="s")
    f = functools.partial(
        pl.kernel,
        mesh=mesh,
        out_type=jax.ShapeDtypeStruct((M, K), jnp.float32),
        scratch_types=[
            pltpu.VMEM((TPW,), jnp.int32),
            pltpu.VMEM((TPW,), jnp.int32),
            pltpu.VMEM((TPW,), jnp.float32),
            pltpu.VMEM((TPW,), jnp.float32),
            pltpu.VMEM((SUB, K), jnp.float32),
            pltpu.VMEM((SUB, K), jnp.float32),
            pltpu.VMEM((SUB, K), jnp.float32),
            pltpu.VMEM((SUB, K), jnp.float32),
            pltpu.SemaphoreType.DMA,
            pltpu.SemaphoreType.DMA,
            pltpu.SemaphoreType.DMA,
            pltpu.SemaphoreType.DMA,
            pltpu.SemaphoreType.DMA,
        ],
    )(_combine_body)
    return f(ye, pos, twT)


def kernel(a1, w1, w2, topk_weights, topk_ids):
    idsT = topk_ids.astype(jnp.int32).T
    twT = topk_weights.T
    pos, be, act, bx = _routing(idsT)
    xs = _dispatch(a1, pos)
    ye = _grouped_gemm(xs, w1, w2, be.reshape(32), act.reshape(32),
                       bx.reshape(32))
    return _combine(ye, pos, twT)
